# Initial kernel scaffold; baseline (speedup 1.0000x reference)
import jax
import jax.numpy as jnp
from jax.experimental import pallas as pl


def kernel(det_value, p, edge_index, logical_idxs, position_ids, time_ids, params):
    raise NotImplementedError("write your pallas kernel here")


# speedup vs baseline: 12.5252x; 12.5252x over previous
import functools
import math

import jax
import jax.numpy as jnp
from jax import lax
from jax.experimental import pallas as pl
from jax.experimental.pallas import tpu as pltpu
from jax.experimental.pallas import tpu_sc as plsc

N = 10240
D = 128
E = 327680
NW = 32
CW = 128
NCHUNK = E // NW // CW
NHALF = NCHUNK // 2
RPT = N // 16

_f32 = jnp.float32

@functools.cache
def _sc_mesh():
    return plsc.VectorSubcoreMesh(core_axis_name="c", subcore_axis_name="s")


def _gelu(x):
    return 0.5 * x * (1.0 + lax.erf(x * (1.0 / math.sqrt(2.0))))


def _emb_body(dv_ref, pos_ref, tm_ref, p_ref, vemb_ref, pemb_ref, temb_ref,
              w1_ref, b1_ref, w2_ref, b2_ref, det_ref, err_ref):
    r = dv_ref.shape[0]
    vemb = vemb_ref[...]
    det = vemb[0:1, :] + dv_ref[...].astype(_f32) * (vemb[1:2, :] - vemb[0:1, :])
    poh = (pos_ref[...] == lax.broadcasted_iota(jnp.int32, (r, 32), 1)).astype(_f32)
    det += jnp.dot(poh, pemb_ref[...], preferred_element_type=_f32)
    toh = (tm_ref[...] == lax.broadcasted_iota(jnp.int32, (r, 16), 1)).astype(_f32)
    det += jnp.dot(toh, temb_ref[...], preferred_element_type=_f32)
    det_ref[...] = det
    x = jnp.log(p_ref[...] * (1.0 / 0.005))
    h = _gelu(x * w1_ref[...] + b1_ref[...])
    err_ref[...] = jnp.dot(h, w2_ref[...], preferred_element_type=_f32) + b2_ref[...]


def _proj_body(x_ref, w_ref, b_ref, o_ref):
    o_ref[...] = jnp.maximum(
        jnp.dot(x_ref[...], w_ref[...], preferred_element_type=_f32) + b_ref[...], 0.0)


def _sageb_body(s_ref, cnt_ref, xd_ref, wl_ref, bl_ref, wr_ref, o_ref):
    s = s_ref[0] + s_ref[1]
    inv = 1.0 / jnp.maximum(cnt_ref[0] + cnt_ref[1], 1.0)
    mean = s * inv
    out = (jnp.dot(mean, wl_ref[...], preferred_element_type=_f32) + bl_ref[...]
           + jnp.dot(xd_ref[...], wr_ref[...], preferred_element_type=_f32))
    nrm = jnp.sqrt(jnp.sum(out * out, axis=1, keepdims=True))
    o_ref[...] = out / jnp.maximum(nrm, 1e-12)


def _clf_body(x_ref, w1_ref, b1_ref, w2_ref, b2_ref, o_ref):
    h = _gelu(jnp.dot(x_ref[...], w1_ref[...], preferred_element_type=_f32) + b1_ref[...])
    o_ref[...] = jnp.dot(h, w2_ref[...], preferred_element_type=_f32) + b2_ref[...]


def _logical_body(idx_ref, x_ref, tw1_ref, tb1_ref, tw2_ref, tb2_ref,
                  lw1_ref, lb1_ref, lw2_ref, lb2_ref, o_ref):
    nl = idx_ref.shape[0]
    oh = (idx_ref[...] == lax.broadcasted_iota(jnp.int32, (nl, 1280), 1)).astype(_f32)
    rel = jnp.dot(oh, x_ref[...], preferred_element_type=_f32)
    t = _gelu(jnp.dot(rel, tw1_ref[...], preferred_element_type=_f32) + tb1_ref[...])
    t = jnp.dot(t, tw2_ref[...], preferred_element_type=_f32) + tb2_ref[...]
    ssum = jnp.sum(t, axis=0, keepdims=True) * (1.0 / math.sqrt(nl))
    lg = _gelu(jnp.dot(ssum, lw1_ref[...], preferred_element_type=_f32) + lb1_ref[...])
    lg = jnp.dot(lg, lw2_ref[...], preferred_element_type=_f32) + lb2_ref[...]
    o_ref[...] = jnp.broadcast_to(lg[None], (1, 1, D))


def _full(shape):
    return pl.BlockSpec(shape, lambda *a: tuple(0 for _ in shape))


_RB = 1024


def _tc_embeddings(dv, pos, tm, p, vemb, pemb, temb, w1, b1, w2, b2):
    grid = (N // _RB,)
    rows = pl.BlockSpec((_RB, 1), lambda i: (i, 0))
    return pl.pallas_call(
        _emb_body,
        grid=grid,
        in_specs=[rows, rows, rows, rows, _full((2, D)), _full((32, D)),
                  _full((16, D)), _full((1, 256)), _full((1, 256)),
                  _full((256, D)), _full((1, D))],
        out_specs=[pl.BlockSpec((_RB, D), lambda i: (i, 0)),
                   pl.BlockSpec((_RB, D), lambda i: (i, 0))],
        out_shape=[jax.ShapeDtypeStruct((N, D), _f32),
                   jax.ShapeDtypeStruct((N, D), _f32)],
    )(dv, pos, tm, p, vemb, pemb, temb, w1, b1, w2, b2)


def _tc_proj(x, w, b):
    return pl.pallas_call(
        _proj_body,
        grid=(N // _RB,),
        in_specs=[pl.BlockSpec((_RB, D), lambda i: (i, 0)), _full((D, D)),
                  _full((1, D))],
        out_specs=pl.BlockSpec((_RB, D), lambda i: (i, 0)),
        out_shape=jax.ShapeDtypeStruct((N, D), _f32),
    )(x, w, b)


def _tc_sageb(s, cnt, xd, wl, bl, wr):
    return pl.pallas_call(
        _sageb_body,
        grid=(N // _RB,),
        in_specs=[pl.BlockSpec((2, _RB, D), lambda i: (0, i, 0)),
                  pl.BlockSpec((2, _RB, 1), lambda i: (0, i, 0)),
                  pl.BlockSpec((_RB, D), lambda i: (i, 0)),
                  _full((D, D)), _full((1, D)), _full((D, D))],
        out_specs=pl.BlockSpec((_RB, D), lambda i: (i, 0)),
        out_shape=jax.ShapeDtypeStruct((N, D), _f32),
    )(s, cnt, xd, wl, bl, wr)


def _tc_classifier(x, w1, b1, w2, b2):
    return pl.pallas_call(
        _clf_body,
        grid=(N // _RB,),
        in_specs=[pl.BlockSpec((_RB, D), lambda i: (i, 0)), _full((D, 256)),
                  _full((1, 256)), _full((256, 1)), _full((1, 1))],
        out_specs=pl.BlockSpec((_RB, 1), lambda i: (i, 0)),
        out_shape=jax.ShapeDtypeStruct((N, 1), _f32),
    )(x, w1, b1, w2, b2)


def _tc_logical(idx, x, tw1, tb1, tw2, tb2, lw1, lb1, lw2, lb2):
    out = pl.pallas_call(
        _logical_body,
        grid=(8,),
        in_specs=[_full((64, 1)),
                  pl.BlockSpec((1280, D), lambda b: (b, 0)),
                  _full((D, 256)), _full((1, 256)), _full((256, D)),
                  _full((1, D)), _full((D, 256)), _full((1, 256)),
                  _full((256, 1)), _full((1, 1))],
        out_specs=pl.BlockSpec((1, 1, D), lambda b: (b, 0, 0)),
        out_shape=jax.ShapeDtypeStruct((8, 1, D), _f32),
    )(idx, x, tw1, tb1, tw2, tb2, lw1, lb1, lw2, lb2)
    return out[:, 0, 0:1]


def _sc_counts(didx):
    @functools.partial(
        pl.kernel,
        mesh=_sc_mesh(),
        out_type=jax.ShapeDtypeStruct((2, 2, N, D), _f32),
        scratch_types=[
            pltpu.VMEM((NCHUNK, CW), jnp.int32),
            pltpu.VMEM((CW, D), _f32),
            pltpu.VMEM_SHARED((N, D), _f32),
        ],
    )
    def cnts(di_hbm, out_hbm, i_v, ones_v, acc):
        cid = lax.axis_index("c")
        sid = lax.axis_index("s")
        wid = cid * 16 + sid
        base = sid * RPT

        def fill(val):
            def body(i, _):
                ones_v[i // 8, pl.ds((i % 8) * 16, 16)] = jnp.full((16,), val, _f32)
                return 0
            lax.fori_loop(0, CW * 8, body, 0)

        for d in range(2):
            fill(0.0)

            def zacc(k, _):
                pltpu.sync_copy(ones_v, acc.at[pl.ds(base + k * CW, CW)])
                return 0
            lax.fori_loop(0, RPT // CW, zacc, 0)
            fill(1.0)
            pltpu.sync_copy(di_hbm.at[d, wid], i_v)
            plsc.subcore_barrier()

            def step(j, _):
                pltpu.sync_copy(ones_v, acc.at[i_v.at[j]], add=True)
                return 0
            lax.fori_loop(0, NCHUNK, step, 0)
            plsc.subcore_barrier()

            def wout(k, _):
                st = base + k * CW
                pltpu.sync_copy(acc.at[pl.ds(st, CW)], ones_v)
                pltpu.sync_copy(ones_v, out_hbm.at[d, cid, pl.ds(st, CW)])
                return 0
            lax.fori_loop(0, RPT // CW, wout, 0)
            plsc.subcore_barrier()

    return cnts(didx)


_NBUF = 2


@functools.cache
def _agg_kernel():
    @functools.partial(
        pl.kernel,
        mesh=_sc_mesh(),
        out_type=jax.ShapeDtypeStruct((2, N, D), _f32),
        scratch_types=[
            pltpu.VMEM((NHALF, CW), jnp.int32),
            pltpu.VMEM((NHALF, CW), jnp.int32),
            pltpu.VMEM((CW, D), _f32),
            pltpu.VMEM((CW, D), _f32),
            pltpu.VMEM_SHARED((N, D), _f32),
            pltpu.SemaphoreType.DMA,
            pltpu.SemaphoreType.DMA,
        ],
    )
    def agg(h_hbm, g_hbm, s_hbm, out_hbm, g_v, s_v, r0, r1, acc, m0, m1):
        bufs = (r0, r1)
        sems = (m0, m1)
        cid = lax.axis_index("c")
        sid = lax.axis_index("s")
        wid = cid * 16 + sid
        base = sid * RPT

        def zrow(i, _):
            r0[i // 8, pl.ds((i % 8) * 16, 16)] = jnp.zeros((16,), _f32)
            return 0
        lax.fori_loop(0, CW * 8, zrow, 0)

        def zacc(k, _):
            pltpu.sync_copy(r0, acc.at[pl.ds(base + k * CW, CW)])
            return 0
        lax.fori_loop(0, RPT // CW, zacc, 0)
        plsc.subcore_barrier()

        for half in range(2):
            pltpu.sync_copy(g_hbm.at[wid, pl.ds(half * NHALF, NHALF)], g_v)
            pltpu.sync_copy(s_hbm.at[wid, pl.ds(half * NHALF, NHALF)], s_v)
            for b in range(_NBUF):
                pltpu.async_copy(h_hbm.at[g_v.at[b]], bufs[b], sems[b])

            def step(i, _):
                for b in range(_NBUF):
                    j = i * _NBUF + b
                    pltpu.make_async_copy(h_hbm.at[g_v.at[j]], bufs[b], sems[b]).wait()
                    pltpu.sync_copy(bufs[b], acc.at[s_v.at[j]], add=True)

                    @pl.when(i < (NHALF // _NBUF) - 1)
                    def _():
                        pltpu.async_copy(h_hbm.at[g_v.at[j + _NBUF]], bufs[b], sems[b])
                return 0
            lax.fori_loop(0, NHALF // _NBUF, step, 0)
        plsc.subcore_barrier()

        def wout(k, _):
            st = base + k * CW
            pltpu.sync_copy(acc.at[pl.ds(st, CW)], r0)
            pltpu.sync_copy(r0, out_hbm.at[cid, pl.ds(st, CW)])
            return 0
        lax.fori_loop(0, RPT // CW, wout, 0)

    return agg


def _sc_agg(h, gidx, sidx):
    return _agg_kernel()(h, gidx, sidx)


def kernel(det_value, p, edge_index, logical_idxs, position_ids, time_ids, params):
    dv = det_value.reshape(-1, 1).astype(jnp.int32)
    pos = position_ids.reshape(-1, 1).astype(jnp.int32)
    tm = time_ids.reshape(-1, 1).astype(jnp.int32)
    ei = edge_index.astype(jnp.int32)
    src = ei[0].reshape(NW, NCHUNK, CW)
    dst = ei[1].reshape(NW, NCHUNK, CW)
    li = logical_idxs.reshape(64, 1).astype(jnp.int32)
    prm = params

    def b2d(v):
        return v.reshape(1, -1)

    pemb = jnp.pad(prm['position_emb'], ((0, 8), (0, 0)))
    temb = jnp.pad(prm['time_emb'], ((0, 5), (0, 0)))

    det_emb, err_emb = _tc_embeddings(
        dv, pos, tm, p, prm['det_value_emb'], pemb, temb,
        prm['p_mlp']['W1'], b2d(prm['p_mlp']['b1']),
        prm['p_mlp']['W2'], b2d(prm['p_mlp']['b2']))

    cnts = _sc_counts(jnp.stack([dst, src]))
    cnt_err = cnts[0, :, :, 0:1]
    cnt_det = cnts[1, :, :, 0:1]

    def sage(x_src, x_dst, g, s, cnt, sp):
        h = _tc_proj(x_src, sp['proj_W'], b2d(sp['proj_b']))
        part = _sc_agg(h, g, s)
        return _tc_sageb(part, cnt, x_dst, sp['lin_l_W'], b2d(sp['lin_l_b']),
                         sp['lin_r_W'])

    for i in range(3):
        err_emb = sage(det_emb, err_emb, src, dst, cnt_err, prm['det_to_error'][i])
        det_emb = sage(err_emb, det_emb, dst, src, cnt_det, prm['error_to_det'][i])
    err_emb = sage(det_emb, err_emb, src, dst, cnt_err, prm['final_det_to_err'])

    errs = _tc_classifier(err_emb, prm['classifier']['W1'], b2d(prm['classifier']['b1']),
                          prm['classifier']['W2'], b2d(prm['classifier']['b2']))
    logical = _tc_logical(li, err_emb,
                          prm['error_transform']['W1'], b2d(prm['error_transform']['b1']),
                          prm['error_transform']['W2'], b2d(prm['error_transform']['b2']),
                          prm['logical_pred']['W1'], b2d(prm['logical_pred']['b1']),
                          prm['logical_pred']['W2'], b2d(prm['logical_pred']['b2']))
    return (errs, logical)


# speedup vs baseline: 13.1431x; 1.0493x over previous
import functools
import math

import jax
import jax.numpy as jnp
from jax import lax
from jax.experimental import pallas as pl
from jax.experimental.pallas import tpu as pltpu
from jax.experimental.pallas import tpu_sc as plsc

N = 10240
D = 128
E = 327680
NW = 32
CW = 128
NCHUNK = E // NW // CW
NHALF = NCHUNK // 2
RPT = N // 16

_f32 = jnp.float32

@functools.cache
def _sc_mesh():
    return plsc.VectorSubcoreMesh(core_axis_name="c", subcore_axis_name="s")


def _gelu(x):
    return 0.5 * x * (1.0 + lax.erf(x * (1.0 / math.sqrt(2.0))))


def _emb_body(dv_ref, pos_ref, tm_ref, p_ref, vemb_ref, pemb_ref, temb_ref,
              w1_ref, b1_ref, w2_ref, b2_ref, det_ref, err_ref):
    r = dv_ref.shape[0]
    vemb = vemb_ref[...]
    det = vemb[0:1, :] + dv_ref[...].astype(_f32) * (vemb[1:2, :] - vemb[0:1, :])
    poh = (pos_ref[...] == lax.broadcasted_iota(jnp.int32, (r, 32), 1)).astype(_f32)
    det += jnp.dot(poh, pemb_ref[...], preferred_element_type=_f32)
    toh = (tm_ref[...] == lax.broadcasted_iota(jnp.int32, (r, 16), 1)).astype(_f32)
    det += jnp.dot(toh, temb_ref[...], preferred_element_type=_f32)
    det_ref[...] = det
    x = jnp.log(p_ref[...] * (1.0 / 0.005))
    h = _gelu(x * w1_ref[...] + b1_ref[...])
    err_ref[...] = jnp.dot(h, w2_ref[...], preferred_element_type=_f32) + b2_ref[...]


def _proj_body(x_ref, w_ref, b_ref, o_ref):
    o_ref[...] = jnp.maximum(
        jnp.dot(x_ref[...], w_ref[...], preferred_element_type=_f32) + b_ref[...], 0.0)


def _sageb_body(s_ref, cnt_ref, xd_ref, wl_ref, bl_ref, wr_ref, o_ref):
    s = s_ref[0] + s_ref[1]
    inv = 1.0 / jnp.maximum(cnt_ref[0] + cnt_ref[1], 1.0)
    mean = s * inv
    out = (jnp.dot(mean, wl_ref[...], preferred_element_type=_f32) + bl_ref[...]
           + jnp.dot(xd_ref[...], wr_ref[...], preferred_element_type=_f32))
    nrm = jnp.sqrt(jnp.sum(out * out, axis=1, keepdims=True))
    o_ref[...] = out / jnp.maximum(nrm, 1e-12)


def _sageb_norm(s_ref, cnt_ref, xd_ref, wl_ref, bl_ref, wr_ref):
    s = s_ref[0] + s_ref[1]
    inv = 1.0 / jnp.maximum(cnt_ref[0] + cnt_ref[1], 1.0)
    mean = s * inv
    out = (jnp.dot(mean, wl_ref[...], preferred_element_type=_f32) + bl_ref[...]
           + jnp.dot(xd_ref[...], wr_ref[...], preferred_element_type=_f32))
    nrm = jnp.sqrt(jnp.sum(out * out, axis=1, keepdims=True))
    return out / jnp.maximum(nrm, 1e-12)


def _sageb_proj_body(s_ref, cnt_ref, xd_ref, wl_ref, bl_ref, wr_ref,
                     wn_ref, bn_ref, o_ref, h_ref):
    out = _sageb_norm(s_ref, cnt_ref, xd_ref, wl_ref, bl_ref, wr_ref)
    o_ref[...] = out
    h_ref[...] = jnp.maximum(
        jnp.dot(out, wn_ref[...], preferred_element_type=_f32) + bn_ref[...], 0.0)


def _sageb_clf_body(s_ref, cnt_ref, xd_ref, wl_ref, bl_ref, wr_ref,
                    w1_ref, b1_ref, w2_ref, b2_ref, o_ref, e_ref):
    out = _sageb_norm(s_ref, cnt_ref, xd_ref, wl_ref, bl_ref, wr_ref)
    o_ref[...] = out
    h = _gelu(jnp.dot(out, w1_ref[...], preferred_element_type=_f32) + b1_ref[...])
    e_ref[...] = jnp.dot(h, w2_ref[...], preferred_element_type=_f32) + b2_ref[...]


def _clf_body(x_ref, w1_ref, b1_ref, w2_ref, b2_ref, o_ref):
    h = _gelu(jnp.dot(x_ref[...], w1_ref[...], preferred_element_type=_f32) + b1_ref[...])
    o_ref[...] = jnp.dot(h, w2_ref[...], preferred_element_type=_f32) + b2_ref[...]


def _logical_body(idx_ref, x_ref, tw1_ref, tb1_ref, tw2_ref, tb2_ref,
                  lw1_ref, lb1_ref, lw2_ref, lb2_ref, o_ref):
    nl = idx_ref.shape[0]
    oh = (idx_ref[...] == lax.broadcasted_iota(jnp.int32, (nl, 1280), 1)).astype(_f32)
    rel = jnp.dot(oh, x_ref[...], preferred_element_type=_f32)
    t = _gelu(jnp.dot(rel, tw1_ref[...], preferred_element_type=_f32) + tb1_ref[...])
    t = jnp.dot(t, tw2_ref[...], preferred_element_type=_f32) + tb2_ref[...]
    ssum = jnp.sum(t, axis=0, keepdims=True) * (1.0 / math.sqrt(nl))
    lg = _gelu(jnp.dot(ssum, lw1_ref[...], preferred_element_type=_f32) + lb1_ref[...])
    lg = jnp.dot(lg, lw2_ref[...], preferred_element_type=_f32) + lb2_ref[...]
    o_ref[...] = jnp.broadcast_to(lg[None], (1, 1, D))


def _full(shape):
    return pl.BlockSpec(shape, lambda *a: tuple(0 for _ in shape))


_RB = 1024


def _tc_embeddings(dv, pos, tm, p, vemb, pemb, temb, w1, b1, w2, b2):
    grid = (N // _RB,)
    rows = pl.BlockSpec((_RB, 1), lambda i: (i, 0))
    return pl.pallas_call(
        _emb_body,
        grid=grid,
        in_specs=[rows, rows, rows, rows, _full((2, D)), _full((32, D)),
                  _full((16, D)), _full((1, 256)), _full((1, 256)),
                  _full((256, D)), _full((1, D))],
        out_specs=[pl.BlockSpec((_RB, D), lambda i: (i, 0)),
                   pl.BlockSpec((_RB, D), lambda i: (i, 0))],
        out_shape=[jax.ShapeDtypeStruct((N, D), _f32),
                   jax.ShapeDtypeStruct((N, D), _f32)],
    )(dv, pos, tm, p, vemb, pemb, temb, w1, b1, w2, b2)


def _tc_proj(x, w, b):
    return pl.pallas_call(
        _proj_body,
        grid=(N // _RB,),
        in_specs=[pl.BlockSpec((_RB, D), lambda i: (i, 0)), _full((D, D)),
                  _full((1, D))],
        out_specs=pl.BlockSpec((_RB, D), lambda i: (i, 0)),
        out_shape=jax.ShapeDtypeStruct((N, D), _f32),
    )(x, w, b)


def _tc_sageb(s, cnt, xd, wl, bl, wr):
    return pl.pallas_call(
        _sageb_body,
        grid=(N // _RB,),
        in_specs=[pl.BlockSpec((2, _RB, D), lambda i: (0, i, 0)),
                  pl.BlockSpec((2, _RB, 1), lambda i: (0, i, 0)),
                  pl.BlockSpec((_RB, D), lambda i: (i, 0)),
                  _full((D, D)), _full((1, D)), _full((D, D))],
        out_specs=pl.BlockSpec((_RB, D), lambda i: (i, 0)),
        out_shape=jax.ShapeDtypeStruct((N, D), _f32),
    )(s, cnt, xd, wl, bl, wr)


def _tc_sageb_proj(sm, cnt, xd, wl, bl, wr, wn, bn):
    rb = pl.BlockSpec((_RB, D), lambda i: (i, 0))
    return pl.pallas_call(
        _sageb_proj_body,
        grid=(N // _RB,),
        in_specs=[pl.BlockSpec((2, _RB, D), lambda i: (0, i, 0)),
                  pl.BlockSpec((2, _RB, 1), lambda i: (0, i, 0)),
                  rb, _full((D, D)), _full((1, D)), _full((D, D)),
                  _full((D, D)), _full((1, D))],
        out_specs=[rb, rb],
        out_shape=[jax.ShapeDtypeStruct((N, D), _f32),
                   jax.ShapeDtypeStruct((N, D), _f32)],
    )(sm, cnt, xd, wl, bl, wr, wn, bn)


def _tc_sageb_clf(sm, cnt, xd, wl, bl, wr, w1, b1, w2, b2):
    rb = pl.BlockSpec((_RB, D), lambda i: (i, 0))
    return pl.pallas_call(
        _sageb_clf_body,
        grid=(N // _RB,),
        in_specs=[pl.BlockSpec((2, _RB, D), lambda i: (0, i, 0)),
                  pl.BlockSpec((2, _RB, 1), lambda i: (0, i, 0)),
                  rb, _full((D, D)), _full((1, D)), _full((D, D)),
                  _full((D, 256)), _full((1, 256)), _full((256, 1)),
                  _full((1, 1))],
        out_specs=[rb, pl.BlockSpec((_RB, 1), lambda i: (i, 0))],
        out_shape=[jax.ShapeDtypeStruct((N, D), _f32),
                   jax.ShapeDtypeStruct((N, 1), _f32)],
    )(sm, cnt, xd, wl, bl, wr, w1, b1, w2, b2)


def _tc_classifier(x, w1, b1, w2, b2):
    return pl.pallas_call(
        _clf_body,
        grid=(N // _RB,),
        in_specs=[pl.BlockSpec((_RB, D), lambda i: (i, 0)), _full((D, 256)),
                  _full((1, 256)), _full((256, 1)), _full((1, 1))],
        out_specs=pl.BlockSpec((_RB, 1), lambda i: (i, 0)),
        out_shape=jax.ShapeDtypeStruct((N, 1), _f32),
    )(x, w1, b1, w2, b2)


def _tc_logical(idx, x, tw1, tb1, tw2, tb2, lw1, lb1, lw2, lb2):
    out = pl.pallas_call(
        _logical_body,
        grid=(8,),
        in_specs=[_full((64, 1)),
                  pl.BlockSpec((1280, D), lambda b: (b, 0)),
                  _full((D, 256)), _full((1, 256)), _full((256, D)),
                  _full((1, D)), _full((D, 256)), _full((1, 256)),
                  _full((256, 1)), _full((1, 1))],
        out_specs=pl.BlockSpec((1, 1, D), lambda b: (b, 0, 0)),
        out_shape=jax.ShapeDtypeStruct((8, 1, D), _f32),
    )(idx, x, tw1, tb1, tw2, tb2, lw1, lb1, lw2, lb2)
    return out[:, 0, 0:1]


def _sc_counts(didx):
    @functools.partial(
        pl.kernel,
        mesh=_sc_mesh(),
        out_type=jax.ShapeDtypeStruct((2, 2, N, D), _f32),
        scratch_types=[
            pltpu.VMEM((NCHUNK, CW), jnp.int32),
            pltpu.VMEM((CW, D), _f32),
            pltpu.VMEM_SHARED((N, D), _f32),
        ],
    )
    def cnts(di_hbm, out_hbm, i_v, ones_v, acc):
        cid = lax.axis_index("c")
        sid = lax.axis_index("s")
        wid = cid * 16 + sid
        base = sid * RPT

        def fill(val):
            def body(i, _):
                ones_v[i // 8, pl.ds((i % 8) * 16, 16)] = jnp.full((16,), val, _f32)
                return 0
            lax.fori_loop(0, CW * 8, body, 0)

        for d in range(2):
            fill(0.0)

            def zacc(k, _):
                pltpu.sync_copy(ones_v, acc.at[pl.ds(base + k * CW, CW)])
                return 0
            lax.fori_loop(0, RPT // CW, zacc, 0)
            fill(1.0)
            pltpu.sync_copy(di_hbm.at[d, wid], i_v)
            plsc.subcore_barrier()

            def step(j, _):
                pltpu.sync_copy(ones_v, acc.at[i_v.at[j]], add=True)
                return 0
            lax.fori_loop(0, NCHUNK, step, 0)
            plsc.subcore_barrier()

            def wout(k, _):
                st = base + k * CW
                pltpu.sync_copy(acc.at[pl.ds(st, CW)], ones_v)
                pltpu.sync_copy(ones_v, out_hbm.at[d, cid, pl.ds(st, CW)])
                return 0
            lax.fori_loop(0, RPT // CW, wout, 0)
            plsc.subcore_barrier()

    return cnts(didx)


_NBUF = 2


@functools.cache
def _agg_kernel():
    @functools.partial(
        pl.kernel,
        mesh=_sc_mesh(),
        out_type=jax.ShapeDtypeStruct((2, N, D), _f32),
        scratch_types=[
            pltpu.VMEM((NHALF, CW), jnp.int32),
            pltpu.VMEM((NHALF, CW), jnp.int32),
            pltpu.VMEM((CW, D), _f32),
            pltpu.VMEM((CW, D), _f32),
            pltpu.VMEM_SHARED((N, D), _f32),
            pltpu.SemaphoreType.DMA,
            pltpu.SemaphoreType.DMA,
        ],
    )
    def agg(h_hbm, g_hbm, s_hbm, out_hbm, g_v, s_v, r0, r1, acc, m0, m1):
        bufs = (r0, r1)
        sems = (m0, m1)
        cid = lax.axis_index("c")
        sid = lax.axis_index("s")
        wid = cid * 16 + sid
        base = sid * RPT

        def zrow(i, _):
            r0[i // 8, pl.ds((i % 8) * 16, 16)] = jnp.zeros((16,), _f32)
            return 0
        lax.fori_loop(0, CW * 8, zrow, 0)

        def zacc(k, _):
            pltpu.sync_copy(r0, acc.at[pl.ds(base + k * CW, CW)])
            return 0
        lax.fori_loop(0, RPT // CW, zacc, 0)
        plsc.subcore_barrier()

        for half in range(2):
            pltpu.sync_copy(g_hbm.at[wid, pl.ds(half * NHALF, NHALF)], g_v)
            pltpu.sync_copy(s_hbm.at[wid, pl.ds(half * NHALF, NHALF)], s_v)
            for b in range(_NBUF):
                pltpu.async_copy(h_hbm.at[g_v.at[b]], bufs[b], sems[b])

            def step(i, _):
                for b in range(_NBUF):
                    j = i * _NBUF + b
                    pltpu.make_async_copy(h_hbm.at[g_v.at[j]], bufs[b], sems[b]).wait()
                    pltpu.sync_copy(bufs[b], acc.at[s_v.at[j]], add=True)

                    @pl.when(i < (NHALF // _NBUF) - 1)
                    def _():
                        pltpu.async_copy(h_hbm.at[g_v.at[j + _NBUF]], bufs[b], sems[b])
                return 0
            lax.fori_loop(0, NHALF // _NBUF, step, 0)
        plsc.subcore_barrier()

        def wout(k, _):
            st = base + k * CW
            pltpu.sync_copy(acc.at[pl.ds(st, CW)], r0)
            pltpu.sync_copy(r0, out_hbm.at[cid, pl.ds(st, CW)])
            return 0
        lax.fori_loop(0, RPT // CW, wout, 0)

    return agg


def _sc_agg(h, gidx, sidx):
    return _agg_kernel()(h, gidx, sidx)


def kernel(det_value, p, edge_index, logical_idxs, position_ids, time_ids, params):
    dv = det_value.reshape(-1, 1).astype(jnp.int32)
    pos = position_ids.reshape(-1, 1).astype(jnp.int32)
    tm = time_ids.reshape(-1, 1).astype(jnp.int32)
    ei = edge_index.astype(jnp.int32)
    src = ei[0].reshape(NW, NCHUNK, CW)
    dst = ei[1].reshape(NW, NCHUNK, CW)
    li = logical_idxs.reshape(64, 1).astype(jnp.int32)
    prm = params

    def b2d(v):
        return v.reshape(1, -1)

    pemb = jnp.pad(prm['position_emb'], ((0, 8), (0, 0)))
    temb = jnp.pad(prm['time_emb'], ((0, 5), (0, 0)))

    det_emb, err_emb = _tc_embeddings(
        dv, pos, tm, p, prm['det_value_emb'], pemb, temb,
        prm['p_mlp']['W1'], b2d(prm['p_mlp']['b1']),
        prm['p_mlp']['W2'], b2d(prm['p_mlp']['b2']))

    cnts = _sc_counts(jnp.stack([dst, src]))
    cnt_err = cnts[0, :, :, 0:1]
    cnt_det = cnts[1, :, :, 0:1]

    layers = []
    for i in range(3):
        layers.append((prm['det_to_error'][i], src, dst, cnt_err))
        layers.append((prm['error_to_det'][i], dst, src, cnt_det))
    layers.append((prm['final_det_to_err'], src, dst, cnt_err))

    h = _tc_proj(det_emb, layers[0][0]['proj_W'], b2d(layers[0][0]['proj_b']))
    emb = [det_emb, err_emb]
    for ly, (sp, g, sdx, cnt) in enumerate(layers):
        part = _sc_agg(h, g, sdx)
        xd = emb[(ly + 1) % 2]
        if ly < 6:
            nsp = layers[ly + 1][0]
            newe, h = _tc_sageb_proj(part, cnt, xd, sp['lin_l_W'],
                                     b2d(sp['lin_l_b']), sp['lin_r_W'],
                                     nsp['proj_W'], b2d(nsp['proj_b']))
            emb[(ly + 1) % 2] = newe
        else:
            err_emb, errs = _tc_sageb_clf(
                part, cnt, xd, sp['lin_l_W'], b2d(sp['lin_l_b']), sp['lin_r_W'],
                prm['classifier']['W1'], b2d(prm['classifier']['b1']),
                prm['classifier']['W2'], b2d(prm['classifier']['b2']))
    logical = _tc_logical(li, err_emb,
                          prm['error_transform']['W1'], b2d(prm['error_transform']['b1']),
                          prm['error_transform']['W2'], b2d(prm['error_transform']['b2']),
                          prm['logical_pred']['W1'], b2d(prm['logical_pred']['b1']),
                          prm['logical_pred']['W2'], b2d(prm['logical_pred']['b2']))
    return (errs, logical)


# speedup vs baseline: 13.1528x; 1.0007x over previous
import functools
import math

import jax
import jax.numpy as jnp
from jax import lax
from jax.experimental import pallas as pl
from jax.experimental.pallas import tpu as pltpu
from jax.experimental.pallas import tpu_sc as plsc

N = 10240
D = 128
E = 327680
NW = 32
CW = 128
NCHUNK = E // NW // CW
NHALF = NCHUNK // 2
RPT = N // 16

_f32 = jnp.float32

@functools.cache
def _sc_mesh():
    return plsc.VectorSubcoreMesh(core_axis_name="c", subcore_axis_name="s")


def _gelu(x):
    return 0.5 * x * (1.0 + lax.erf(x * (1.0 / math.sqrt(2.0))))


def _emb_body(dv_ref, pos_ref, tm_ref, p_ref, vemb_ref, pemb_ref, temb_ref,
              w1_ref, b1_ref, w2_ref, b2_ref, det_ref, err_ref):
    r = dv_ref.shape[0]
    vemb = vemb_ref[...]
    det = vemb[0:1, :] + dv_ref[...].astype(_f32) * (vemb[1:2, :] - vemb[0:1, :])
    poh = (pos_ref[...] == lax.broadcasted_iota(jnp.int32, (r, 32), 1)).astype(_f32)
    det += jnp.dot(poh, pemb_ref[...], preferred_element_type=_f32)
    toh = (tm_ref[...] == lax.broadcasted_iota(jnp.int32, (r, 16), 1)).astype(_f32)
    det += jnp.dot(toh, temb_ref[...], preferred_element_type=_f32)
    det_ref[...] = det
    x = jnp.log(p_ref[...] * (1.0 / 0.005))
    h = _gelu(x * w1_ref[...] + b1_ref[...])
    err_ref[...] = jnp.dot(h, w2_ref[...], preferred_element_type=_f32) + b2_ref[...]


def _proj_body(x_ref, w_ref, b_ref, o_ref):
    o_ref[...] = jnp.maximum(
        jnp.dot(x_ref[...], w_ref[...], preferred_element_type=_f32) + b_ref[...], 0.0)


def _sageb_body(s_ref, cnt_ref, xd_ref, wl_ref, bl_ref, wr_ref, o_ref):
    s = s_ref[0] + s_ref[1]
    inv = 1.0 / jnp.maximum(cnt_ref[0] + cnt_ref[1], 1.0)
    mean = s * inv
    out = (jnp.dot(mean, wl_ref[...], preferred_element_type=_f32) + bl_ref[...]
           + jnp.dot(xd_ref[...], wr_ref[...], preferred_element_type=_f32))
    nrm = jnp.sqrt(jnp.sum(out * out, axis=1, keepdims=True))
    o_ref[...] = out / jnp.maximum(nrm, 1e-12)


def _sageb_norm(s_ref, cnt_ref, xd_ref, wl_ref, bl_ref, wr_ref):
    s = s_ref[0] + s_ref[1]
    inv = 1.0 / jnp.maximum(cnt_ref[0] + cnt_ref[1], 1.0)
    mean = s * inv
    out = (jnp.dot(mean, wl_ref[...], preferred_element_type=_f32) + bl_ref[...]
           + jnp.dot(xd_ref[...], wr_ref[...], preferred_element_type=_f32))
    nrm = jnp.sqrt(jnp.sum(out * out, axis=1, keepdims=True))
    return out / jnp.maximum(nrm, 1e-12)


def _sageb_proj_body(s_ref, cnt_ref, xd_ref, wl_ref, bl_ref, wr_ref,
                     wn_ref, bn_ref, o_ref, h_ref):
    out = _sageb_norm(s_ref, cnt_ref, xd_ref, wl_ref, bl_ref, wr_ref)
    o_ref[...] = out
    h_ref[...] = jnp.maximum(
        jnp.dot(out, wn_ref[...], preferred_element_type=_f32) + bn_ref[...], 0.0)


def _sageb_clf_body(s_ref, cnt_ref, xd_ref, wl_ref, bl_ref, wr_ref,
                    w1_ref, b1_ref, w2_ref, b2_ref, o_ref, e_ref):
    out = _sageb_norm(s_ref, cnt_ref, xd_ref, wl_ref, bl_ref, wr_ref)
    o_ref[...] = out
    h = _gelu(jnp.dot(out, w1_ref[...], preferred_element_type=_f32) + b1_ref[...])
    e_ref[...] = jnp.dot(h, w2_ref[...], preferred_element_type=_f32) + b2_ref[...]


def _clf_body(x_ref, w1_ref, b1_ref, w2_ref, b2_ref, o_ref):
    h = _gelu(jnp.dot(x_ref[...], w1_ref[...], preferred_element_type=_f32) + b1_ref[...])
    o_ref[...] = jnp.dot(h, w2_ref[...], preferred_element_type=_f32) + b2_ref[...]


def _logical_body(idx_ref, x_ref, tw1_ref, tb1_ref, tw2_ref, tb2_ref,
                  lw1_ref, lb1_ref, lw2_ref, lb2_ref, o_ref):
    nl = idx_ref.shape[0]
    oh = (idx_ref[...] == lax.broadcasted_iota(jnp.int32, (nl, 1280), 1)).astype(_f32)
    rel = jnp.dot(oh, x_ref[...], preferred_element_type=_f32)
    t = _gelu(jnp.dot(rel, tw1_ref[...], preferred_element_type=_f32) + tb1_ref[...])
    t = jnp.dot(t, tw2_ref[...], preferred_element_type=_f32) + tb2_ref[...]
    ssum = jnp.sum(t, axis=0, keepdims=True) * (1.0 / math.sqrt(nl))
    lg = _gelu(jnp.dot(ssum, lw1_ref[...], preferred_element_type=_f32) + lb1_ref[...])
    lg = jnp.dot(lg, lw2_ref[...], preferred_element_type=_f32) + lb2_ref[...]
    o_ref[...] = jnp.broadcast_to(lg[None], (1, 1, D))


def _full(shape):
    return pl.BlockSpec(shape, lambda *a: tuple(0 for _ in shape))


_RB = 1024


def _tc_embeddings(dv, pos, tm, p, vemb, pemb, temb, w1, b1, w2, b2):
    grid = (N // _RB,)
    rows = pl.BlockSpec((_RB, 1), lambda i: (i, 0))
    return pl.pallas_call(
        _emb_body,
        grid=grid,
        in_specs=[rows, rows, rows, rows, _full((2, D)), _full((32, D)),
                  _full((16, D)), _full((1, 256)), _full((1, 256)),
                  _full((256, D)), _full((1, D))],
        out_specs=[pl.BlockSpec((_RB, D), lambda i: (i, 0)),
                   pl.BlockSpec((_RB, D), lambda i: (i, 0))],
        out_shape=[jax.ShapeDtypeStruct((N, D), _f32),
                   jax.ShapeDtypeStruct((N, D), _f32)],
    )(dv, pos, tm, p, vemb, pemb, temb, w1, b1, w2, b2)


def _tc_proj(x, w, b):
    return pl.pallas_call(
        _proj_body,
        grid=(N // _RB,),
        in_specs=[pl.BlockSpec((_RB, D), lambda i: (i, 0)), _full((D, D)),
                  _full((1, D))],
        out_specs=pl.BlockSpec((_RB, D), lambda i: (i, 0)),
        out_shape=jax.ShapeDtypeStruct((N, D), _f32),
    )(x, w, b)


def _tc_sageb(s, cnt, xd, wl, bl, wr):
    return pl.pallas_call(
        _sageb_body,
        grid=(N // _RB,),
        in_specs=[pl.BlockSpec((2, _RB, D), lambda i: (0, i, 0)),
                  pl.BlockSpec((2, _RB, 1), lambda i: (0, i, 0)),
                  pl.BlockSpec((_RB, D), lambda i: (i, 0)),
                  _full((D, D)), _full((1, D)), _full((D, D))],
        out_specs=pl.BlockSpec((_RB, D), lambda i: (i, 0)),
        out_shape=jax.ShapeDtypeStruct((N, D), _f32),
    )(s, cnt, xd, wl, bl, wr)


def _tc_sageb_proj(sm, cnt, xd, wl, bl, wr, wn, bn):
    rb = pl.BlockSpec((_RB, D), lambda i: (i, 0))
    return pl.pallas_call(
        _sageb_proj_body,
        grid=(N // _RB,),
        in_specs=[pl.BlockSpec((2, _RB, D), lambda i: (0, i, 0)),
                  pl.BlockSpec((2, _RB, 1), lambda i: (0, i, 0)),
                  rb, _full((D, D)), _full((1, D)), _full((D, D)),
                  _full((D, D)), _full((1, D))],
        out_specs=[rb, rb],
        out_shape=[jax.ShapeDtypeStruct((N, D), _f32),
                   jax.ShapeDtypeStruct((N, D), _f32)],
    )(sm, cnt, xd, wl, bl, wr, wn, bn)


def _tc_sageb_clf(sm, cnt, xd, wl, bl, wr, w1, b1, w2, b2):
    rb = pl.BlockSpec((_RB, D), lambda i: (i, 0))
    return pl.pallas_call(
        _sageb_clf_body,
        grid=(N // _RB,),
        in_specs=[pl.BlockSpec((2, _RB, D), lambda i: (0, i, 0)),
                  pl.BlockSpec((2, _RB, 1), lambda i: (0, i, 0)),
                  rb, _full((D, D)), _full((1, D)), _full((D, D)),
                  _full((D, 256)), _full((1, 256)), _full((256, 1)),
                  _full((1, 1))],
        out_specs=[rb, pl.BlockSpec((_RB, 1), lambda i: (i, 0))],
        out_shape=[jax.ShapeDtypeStruct((N, D), _f32),
                   jax.ShapeDtypeStruct((N, 1), _f32)],
    )(sm, cnt, xd, wl, bl, wr, w1, b1, w2, b2)


def _tc_classifier(x, w1, b1, w2, b2):
    return pl.pallas_call(
        _clf_body,
        grid=(N // _RB,),
        in_specs=[pl.BlockSpec((_RB, D), lambda i: (i, 0)), _full((D, 256)),
                  _full((1, 256)), _full((256, 1)), _full((1, 1))],
        out_specs=pl.BlockSpec((_RB, 1), lambda i: (i, 0)),
        out_shape=jax.ShapeDtypeStruct((N, 1), _f32),
    )(x, w1, b1, w2, b2)


def _tc_logical(idx, x, tw1, tb1, tw2, tb2, lw1, lb1, lw2, lb2):
    out = pl.pallas_call(
        _logical_body,
        grid=(8,),
        in_specs=[_full((64, 1)),
                  pl.BlockSpec((1280, D), lambda b: (b, 0)),
                  _full((D, 256)), _full((1, 256)), _full((256, D)),
                  _full((1, D)), _full((D, 256)), _full((1, 256)),
                  _full((256, 1)), _full((1, 1))],
        out_specs=pl.BlockSpec((1, 1, D), lambda b: (b, 0, 0)),
        out_shape=jax.ShapeDtypeStruct((8, 1, D), _f32),
    )(idx, x, tw1, tb1, tw2, tb2, lw1, lb1, lw2, lb2)
    return out[:, 0, 0:1]


def _sc_counts(didx):
    @functools.partial(
        pl.kernel,
        mesh=_sc_mesh(),
        out_type=jax.ShapeDtypeStruct((2, 2, N, D), _f32),
        scratch_types=[
            pltpu.VMEM((NCHUNK, CW), jnp.int32),
            pltpu.VMEM((CW, D), _f32),
            pltpu.VMEM_SHARED((N, D), _f32),
        ],
    )
    def cnts(di_hbm, out_hbm, i_v, ones_v, acc):
        cid = lax.axis_index("c")
        sid = lax.axis_index("s")
        wid = cid * 16 + sid
        base = sid * RPT

        def fill(val):
            def body(i, _):
                ones_v[i // 8, pl.ds((i % 8) * 16, 16)] = jnp.full((16,), val, _f32)
                return 0
            lax.fori_loop(0, CW * 8, body, 0)

        for d in range(2):
            fill(0.0)

            def zacc(k, _):
                pltpu.sync_copy(ones_v, acc.at[pl.ds(base + k * CW, CW)])
                return 0
            lax.fori_loop(0, RPT // CW, zacc, 0)
            fill(1.0)
            pltpu.sync_copy(di_hbm.at[d, wid], i_v)
            plsc.subcore_barrier()

            def step(j, _):
                pltpu.sync_copy(ones_v, acc.at[i_v.at[j]], add=True)
                return 0
            lax.fori_loop(0, NCHUNK, step, 0)
            plsc.subcore_barrier()

            def wout(k, _):
                st = base + k * CW
                pltpu.sync_copy(acc.at[pl.ds(st, CW)], ones_v)
                pltpu.sync_copy(ones_v, out_hbm.at[d, cid, pl.ds(st, CW)])
                return 0
            lax.fori_loop(0, RPT // CW, wout, 0)
            plsc.subcore_barrier()

    return cnts(didx)


@functools.cache
def _agg_kernel():
    @functools.partial(
        pl.kernel,
        mesh=_sc_mesh(),
        out_type=jax.ShapeDtypeStruct((2, N, D), _f32),
        scratch_types=[
            pltpu.VMEM((NHALF, CW), jnp.int32),
            pltpu.VMEM((NHALF, CW), jnp.int32),
            pltpu.VMEM((CW, D), _f32),
            pltpu.VMEM((CW, D), _f32),
            pltpu.VMEM_SHARED((N, D), _f32),
            pltpu.SemaphoreType.DMA,
            pltpu.SemaphoreType.DMA,
        ],
    )
    def agg(h_hbm, g_hbm, s_hbm, out_hbm, g_v, s_v, r0, r1, acc, m0, m1):
        bufs = (r0, r1)
        sems = (m0, m1)
        cid = lax.axis_index("c")
        sid = lax.axis_index("s")
        wid = cid * 16 + sid
        base = sid * RPT

        def zrow(i, _):
            r0[i // 8, pl.ds((i % 8) * 16, 16)] = jnp.zeros((16,), _f32)
            return 0
        lax.fori_loop(0, CW * 8, zrow, 0)

        def zacc(k, _):
            pltpu.sync_copy(r0, acc.at[pl.ds(base + k * CW, CW)])
            return 0
        lax.fori_loop(0, RPT // CW, zacc, 0)
        plsc.subcore_barrier()

        for half in range(2):
            pltpu.sync_copy(g_hbm.at[wid, pl.ds(half * NHALF, NHALF)], g_v)
            pltpu.sync_copy(s_hbm.at[wid, pl.ds(half * NHALF, NHALF)], s_v)
            for b in range(2):
                pltpu.async_copy(h_hbm.at[g_v.at[b]], bufs[b], sems[b])

            def step(i, _):
                for b in range(2):
                    j = i * 2 + b
                    pltpu.make_async_copy(h_hbm.at[g_v.at[j]], bufs[b],
                                          sems[b]).wait()
                    pltpu.sync_copy(bufs[b], acc.at[s_v.at[j]], add=True)

                    @pl.when(j + 2 < NHALF)
                    def _():
                        pltpu.async_copy(h_hbm.at[g_v.at[j + 2]], bufs[b],
                                         sems[b])
                return 0
            lax.fori_loop(0, NHALF // 2, step, 0)
        plsc.subcore_barrier()

        def wout(k, _):
            st = base + k * CW
            pltpu.sync_copy(acc.at[pl.ds(st, CW)], r0)
            pltpu.sync_copy(r0, out_hbm.at[cid, pl.ds(st, CW)])
            return 0
        lax.fori_loop(0, RPT // CW, wout, 0)

    return agg


def _sc_agg(h, gidx, sidx):
    return _agg_kernel()(h, gidx, sidx)


def kernel(det_value, p, edge_index, logical_idxs, position_ids, time_ids, params):
    dv = det_value.reshape(-1, 1).astype(jnp.int32)
    pos = position_ids.reshape(-1, 1).astype(jnp.int32)
    tm = time_ids.reshape(-1, 1).astype(jnp.int32)
    ei = edge_index.astype(jnp.int32)
    src = ei[0].reshape(NW, NCHUNK, CW)
    dst = ei[1].reshape(NW, NCHUNK, CW)
    li = logical_idxs.reshape(64, 1).astype(jnp.int32)
    prm = params

    def b2d(v):
        return v.reshape(1, -1)

    pemb = jnp.pad(prm['position_emb'], ((0, 8), (0, 0)))
    temb = jnp.pad(prm['time_emb'], ((0, 5), (0, 0)))

    det_emb, err_emb = _tc_embeddings(
        dv, pos, tm, p, prm['det_value_emb'], pemb, temb,
        prm['p_mlp']['W1'], b2d(prm['p_mlp']['b1']),
        prm['p_mlp']['W2'], b2d(prm['p_mlp']['b2']))

    cnts = _sc_counts(jnp.stack([dst, src]))
    cnt_err = cnts[0, :, :, 0:1]
    cnt_det = cnts[1, :, :, 0:1]

    layers = []
    for i in range(3):
        layers.append((prm['det_to_error'][i], src, dst, cnt_err))
        layers.append((prm['error_to_det'][i], dst, src, cnt_det))
    layers.append((prm['final_det_to_err'], src, dst, cnt_err))

    h = _tc_proj(det_emb, layers[0][0]['proj_W'], b2d(layers[0][0]['proj_b']))
    emb = [det_emb, err_emb]
    for ly, (sp, g, sdx, cnt) in enumerate(layers):
        part = _sc_agg(h, g, sdx)
        xd = emb[(ly + 1) % 2]
        if ly < 6:
            nsp = layers[ly + 1][0]
            newe, h = _tc_sageb_proj(part, cnt, xd, sp['lin_l_W'],
                                     b2d(sp['lin_l_b']), sp['lin_r_W'],
                                     nsp['proj_W'], b2d(nsp['proj_b']))
            emb[(ly + 1) % 2] = newe
        else:
            err_emb, errs = _tc_sageb_clf(
                part, cnt, xd, sp['lin_l_W'], b2d(sp['lin_l_b']), sp['lin_r_W'],
                prm['classifier']['W1'], b2d(prm['classifier']['b1']),
                prm['classifier']['W2'], b2d(prm['classifier']['b2']))
    logical = _tc_logical(li, err_emb,
                          prm['error_transform']['W1'], b2d(prm['error_transform']['b1']),
                          prm['error_transform']['W2'], b2d(prm['error_transform']['b2']),
                          prm['logical_pred']['W1'], b2d(prm['logical_pred']['b1']),
                          prm['logical_pred']['W2'], b2d(prm['logical_pred']['b2']))
    return (errs, logical)
